# baseline (speedup 1.0000x reference)
import math

import jax
import jax.numpy as jnp
from jax.experimental import pallas as pl
from jax.experimental.pallas import tpu as pltpu

_H_IMG = 128
_W_IMG = 128
_XG = 32
_N = _XG * _XG
_HID = 256
_TP = 4096


def _shift_rows(f, delta, zeros):
    if delta == 0:
        return f
    if delta > 0:
        return jnp.concatenate([f[delta:, :], zeros[:delta, :]], axis=0)
    return jnp.concatenate([zeros[:(-delta), :], f[:delta, :]], axis=0)


def _encode_kernel(x_ref, wc_ref, aux_ref, w1_ref, b1_ref, w2_ref, b2_ref,
                   coord_ref, o_ref):
    M = x_ref.shape[0]
    qi = jax.lax.broadcasted_iota(jnp.int32, (M, 1), 0)
    yidx = qi % _XG
    xidx = (qi // _XG) % _XG
    one = jnp.ones((M, 1), jnp.float32)
    ym = {0: (yidx >= 1).astype(jnp.float32), 1: one,
          2: (yidx < _XG - 1).astype(jnp.float32)}
    xm = {0: (xidx >= 1).astype(jnp.float32), 1: one,
          2: (xidx < _XG - 1).astype(jnp.float32)}
    masks = {(i, j): (None if i == 1 and j == 1 else xm[i] * ym[j])
             for i in range(3) for j in range(3)}

    def masked_shift(arr, zeros, delta, ymask):
        u = _shift_rows(arr, delta, zeros)
        if ymask is not None:
            u = u * ymask
        return u

    def edge(i, j):
        delta = (i - 1) * _XG + (j - 1)
        return delta, masks[(i, j)]

    xt = x_ref[...]
    zeros_x = jnp.zeros_like(xt)
    fa = jnp.zeros((M, wc_ref.shape[2]), jnp.float32)
    for i in range(3):
        for j in range(3):
            delta, ymask = edge(i, j)
            xs = masked_shift(xt, zeros_x, delta, ymask)
            fa = fa + jnp.dot(xs, wc_ref[i * 3 + j],
                              preferred_element_type=jnp.float32)
    f = jax.nn.relu(fa + aux_ref[0:1, :])
    C = f.shape[1]
    zeros_f = jnp.zeros_like(f)
    g1 = jnp.sum(f, axis=1, keepdims=True)
    g2 = jnp.sum(f * f, axis=1, keepdims=True)
    zcol = jnp.zeros_like(g1)

    s1 = jnp.zeros((M, 1), jnp.float32)
    s2 = jnp.zeros((M, 1), jnp.float32)
    for i in range(3):
        for j in range(3):
            delta, ymask = edge(i, j)
            s1 = s1 + masked_shift(g1, zcol, delta, ymask)
            s2 = s2 + masked_shift(g2, zcol, delta, ymask)

    d_inv = 1.0 / (9.0 * C)
    mu = s1 * d_inv
    var = s2 * d_inv - mu * mu
    rinv = jax.lax.rsqrt(var + 1e-5)

    vs = []
    for i in range(3):
        for j in range(3):
            delta, ymask = edge(i, j)
            u = masked_shift(f, zeros_f, delta, ymask)
            vs.append(rinv * (u - mu))
    v = jnp.concatenate(vs, axis=1)
    m = jnp.dot(v, w1_ref[...], preferred_element_type=jnp.float32)

    h = jax.nn.relu(m + b1_ref[...])

    o_all = jax.lax.dot_general(
        w2_ref[...], h, (((0,), (1,)), ((), ())),
        preferred_element_type=jnp.float32) + b2_ref[...]
    o_off = o_all[0:2, :]
    o_sc = o_all[2:4, :]
    o_rot = o_all[4:5, :]
    o_col = o_all[5:8, :]

    tw = aux_ref[1, 0]
    th = aux_ref[1, 1]
    two_factor = aux_ref[1, 2]
    three_off = aux_ref[1, 3]
    xy = coord_ref[...] + jnp.tanh(o_off) * three_off
    cx = 0.5 * (xy[0:1, :] + 1.0) * tw
    cy = 0.5 * (xy[1:2, :] + 1.0) * th
    scale = jax.nn.sigmoid(o_sc) * two_factor
    sx2 = scale[0:1, :] * scale[0:1, :]
    sy2 = scale[1:2, :] * scale[1:2, :]
    theta = jax.nn.sigmoid(o_rot) * (2.0 * math.pi)
    c = jnp.cos(theta)
    s = jnp.sin(theta)
    a = c * c * sx2 + s * s * sy2
    b = c * s * (sx2 - sy2)
    d = s * s * sx2 + c * c * sy2
    det = jnp.maximum(a * d - b * b, 1e-8)
    o_ref[0:1, :] = cx
    o_ref[1:2, :] = cy
    o_ref[2:3, :] = d / det
    o_ref[3:4, :] = -b / det
    o_ref[4:5, :] = a / det
    o_ref[5:8, :] = jnp.tanh(o_col)


def _render_kernel(geo_ref, o_ref, aw_ref, row_ref):
    t = pl.program_id(1)
    geo = geo_ref[...]
    L2E = 1.4426950408889634

    @pl.when(t == 0)
    def _build_tables():
        cx = geo[0:1, :]
        ia = geo[2:3, :]
        ib = geo[3:4, :]
        idd = geo[4:5, :]
        ri = jax.lax.rsqrt(ia)
        sa = (0.5 * L2E) ** 0.5 * ia * ri
        c1 = (0.5 * L2E) ** 0.5 * ib * ri
        mq = -jnp.maximum((0.5 * L2E) * idd - c1 * c1, 0.0)
        row_ref[0:1, :] = c1
        row_ref[1:2, :] = mq
        pxw = jax.lax.broadcasted_iota(
            jnp.int32, (_W_IMG, 1), 0).astype(jnp.float32) + 0.5
        aw_ref[...] = sa * (pxw - cx)

    cy = geo[1:2, :]
    R = _TP // _W_IMG
    pyh = (jax.lax.broadcasted_iota(jnp.int32, (R, 1), 0).astype(jnp.float32)
           + (t * R).astype(jnp.float32) + 0.5)
    dyh = pyh - cy
    k2 = row_ref[0:1, :] * dyh
    bh = (row_ref[1:2, :] * dyh) * dyh
    aw = aw_ref[...][None, :, :]
    u = aw + k2[:, None, :]
    power = bh[:, None, :] - u * u
    alpha = jnp.exp2(power).reshape(_TP, _N)
    acc = jax.lax.dot_general(
        geo[5:8, :], alpha, (((1,), (1,)), ((), ())),
        preferred_element_type=jnp.float32)
    o_ref[0] = jnp.clip(acc, 0.0, 1.0)


def kernel(inp, conv_w, conv_b, off_w1, off_b1, off_w2, off_b2,
           sc_w1, sc_b1, sc_w2, sc_b2, rot_w1, rot_b1, rot_w2, rot_b2,
           col_w1, col_b1, col_w2, col_b2, target_h, target_w):
    f32 = jnp.float32
    B, Cin, h_in, w_in = inp.shape
    C = conv_w.shape[0]

    Cp = 8
    xt = jnp.transpose(inp, (0, 3, 2, 1)).reshape(B, _N, Cin)
    xt = jnp.pad(xt, ((0, 0), (0, 0), (0, Cp - Cin)))
    wc9 = jnp.transpose(conv_w, (3, 2, 1, 0))
    wc9 = jnp.pad(wc9, ((0, 0), (0, 0), (0, Cp - Cin), (0, 0)))
    wc9 = wc9.reshape(9, Cp, C)

    w1p = jnp.concatenate([off_w1, sc_w1, rot_w1, col_w1], axis=1)
    w1p = w1p.reshape(C, 9, 4 * _HID).transpose(1, 0, 2).reshape(9 * C,
                                                                 4 * _HID)
    b1 = jnp.concatenate([off_b1, sc_b1, rot_b1, col_b1]).reshape(1, 4 * _HID)
    w2 = jnp.zeros((4 * _HID, 8), f32)
    w2 = w2.at[0 * _HID:1 * _HID, 0:2].set(off_w2)
    w2 = w2.at[1 * _HID:2 * _HID, 2:4].set(sc_w2)
    w2 = w2.at[2 * _HID:3 * _HID, 4:5].set(rot_w2)
    w2 = w2.at[3 * _HID:4 * _HID, 5:8].set(col_w2)
    b2 = jnp.concatenate([off_b2, sc_b2, rot_b2, col_b2]).reshape(8, 1)

    th_f = jnp.asarray(target_h, f32)
    tw_f = jnp.asarray(target_w, f32)
    factor = jnp.maximum(th_f / h_in, tw_f / w_in)
    off_factor = 2.0 * factor / jnp.maximum(th_f, tw_f)
    scal = jnp.stack([tw_f, th_f, 2.0 * factor, 3.0 * off_factor])
    aux = jnp.stack([conv_b, jnp.pad(scal, (0, C - 4))], axis=0)

    r = 1.0 / _XG
    c1 = -1.0 + r + 2.0 * r * jnp.arange(_XG, dtype=f32)
    coord = jnp.stack(jnp.meshgrid(c1, c1, indexing='ij'), axis=-1)
    coordT = coord.reshape(_N, 2).T
    coordT = jnp.concatenate([coordT] * B, axis=1)

    geo = pl.pallas_call(
        _encode_kernel,
        out_shape=jax.ShapeDtypeStruct((8, B * _N), f32),
    )(xt.reshape(B * _N, Cp), wc9, aux, w1p, b1, w2, b2, coordT)

    n_pix = _H_IMG * _W_IMG
    n_tiles = n_pix // _TP
    out = pl.pallas_call(
        _render_kernel,
        grid=(B, n_tiles),
        in_specs=[
            pl.BlockSpec((8, _N), lambda b, t: (0, b)),
        ],
        out_specs=pl.BlockSpec((1, 3, _TP), lambda b, t: (b, 0, t)),
        out_shape=jax.ShapeDtypeStruct((B, 3, n_pix), f32),
        scratch_shapes=[
            pltpu.VMEM((_W_IMG, _N), f32),
            pltpu.VMEM((8, _N), f32),
        ],
    )(geo)

    return out.reshape(B, 3, _H_IMG, _W_IMG)


# speedup vs baseline: 1.1055x; 1.1055x over previous
import math

import jax
import jax.numpy as jnp
from jax.experimental import pallas as pl
from jax.experimental.pallas import tpu as pltpu

_H_IMG = 128
_W_IMG = 128
_XG = 32
_N = _XG * _XG
_HID = 256
_TP = 4096


def _shift_rows(f, delta, zeros):
    if delta == 0:
        return f
    if delta > 0:
        return jnp.concatenate([f[delta:, :], zeros[:delta, :]], axis=0)
    return jnp.concatenate([zeros[:(-delta), :], f[:delta, :]], axis=0)


def _encode_kernel(p_ref, wc_ref, bc_ref, w1o_ref, w1s_ref, w1r_ref, w1c_ref,
                   b1_ref, w2o_ref, w2s_ref, w2r_ref, w2c_ref, b2_ref,
                   coord_ref, scal_ref, o_ref):
    f = jax.nn.relu(
        jnp.dot(p_ref[0], wc_ref[...], preferred_element_type=jnp.float32)
        + bc_ref[...])
    C = f.shape[1]
    zeros_f = jnp.zeros_like(f)
    g1 = jnp.sum(f, axis=1, keepdims=True)
    g2 = jnp.sum(f * f, axis=1, keepdims=True)
    zcol = jnp.zeros_like(g1)

    yidx = jax.lax.broadcasted_iota(jnp.int32, (_N, 1), 0) % _XG

    def masked_shift(arr, zeros, delta, ymask):
        u = _shift_rows(arr, delta, zeros)
        if ymask is not None:
            u = u * ymask
        return u

    def edge(i, j):
        delta = (i - 1) * _XG + (j - 1)
        if j == 0:
            ymask = (yidx >= 1).astype(jnp.float32)
        elif j == 2:
            ymask = (yidx < _XG - 1).astype(jnp.float32)
        else:
            ymask = None
        return delta, ymask

    s1 = jnp.zeros((_N, 1), jnp.float32)
    s2 = jnp.zeros((_N, 1), jnp.float32)
    for i in range(3):
        for j in range(3):
            delta, ymask = edge(i, j)
            s1 = s1 + masked_shift(g1, zcol, delta, ymask)
            s2 = s2 + masked_shift(g2, zcol, delta, ymask)

    d_inv = 1.0 / (9.0 * C)
    mu = s1 * d_inv
    var = s2 * d_inv - mu * mu
    rinv = jax.lax.rsqrt(var + 1e-5)

    vs = []
    for i in range(3):
        for j in range(3):
            delta, ymask = edge(i, j)
            u = masked_shift(f, zeros_f, delta, ymask)
            vs.append(rinv * (u - mu))
    m = jnp.zeros((_N, 4 * _HID), jnp.float32)
    for ij in range(9):
        w1ij = jnp.concatenate(
            [w1o_ref[:, ij, :], w1s_ref[:, ij, :],
             w1r_ref[:, ij, :], w1c_ref[:, ij, :]], axis=1)
        m = m + jnp.dot(vs[ij], w1ij, preferred_element_type=jnp.float32)

    h = jax.nn.relu(m + b1_ref[...])

    def headT(w2_ref_, b2_col, lo):
        hh = h[:, lo * _HID:(lo + 1) * _HID]
        return jax.lax.dot_general(
            w2_ref_[...], hh, (((0,), (1,)), ((), ())),
            preferred_element_type=jnp.float32) + b2_col

    b2 = b2_ref[...]
    o_off = headT(w2o_ref, b2[0:2, :], 0)
    o_sc = headT(w2s_ref, b2[2:4, :], 1)
    o_rot = headT(w2r_ref, b2[4:5, :], 2)
    o_col = headT(w2c_ref, b2[5:8, :], 3)

    tw = scal_ref[0, 0]
    th = scal_ref[0, 1]
    two_factor = scal_ref[0, 2]
    three_off = scal_ref[0, 3]
    xy = coord_ref[...] + jnp.tanh(o_off) * three_off
    cx = 0.5 * (xy[0:1, :] + 1.0) * tw
    cy = 0.5 * (xy[1:2, :] + 1.0) * th
    scale = jax.nn.sigmoid(o_sc) * two_factor
    sx2 = scale[0:1, :] * scale[0:1, :]
    sy2 = scale[1:2, :] * scale[1:2, :]
    theta = jax.nn.sigmoid(o_rot) * (2.0 * math.pi)
    c = jnp.cos(theta)
    s = jnp.sin(theta)
    a = c * c * sx2 + s * s * sy2
    b = c * s * (sx2 - sy2)
    d = s * s * sx2 + c * c * sy2
    det = jnp.maximum(a * d - b * b, 1e-8)
    o_ref[0, 0:1, :] = cx
    o_ref[0, 1:2, :] = cy
    o_ref[0, 2:3, :] = d / det
    o_ref[0, 3:4, :] = -b / det
    o_ref[0, 4:5, :] = a / det
    o_ref[0, 5:8, :] = jnp.tanh(o_col)


def _render_kernel(geo_ref, o_ref, aw_ref, row_ref):
    t = pl.program_id(1)
    geo = geo_ref[0]
    L2E = 1.4426950408889634

    @pl.when(t == 0)
    def _build_tables():
        cx = geo[0:1, :]
        ia = geo[2:3, :]
        ib = geo[3:4, :]
        idd = geo[4:5, :]
        ri = jax.lax.rsqrt(ia)
        sa = (0.5 * L2E) ** 0.5 * ia * ri
        c1 = (0.5 * L2E) ** 0.5 * ib * ri
        mq = -jnp.maximum((0.5 * L2E) * idd - c1 * c1, 0.0)
        row_ref[0:1, :] = c1
        row_ref[1:2, :] = mq
        pxw = jax.lax.broadcasted_iota(
            jnp.int32, (_W_IMG, 1), 0).astype(jnp.float32) + 0.5
        aw_ref[...] = sa * (pxw - cx)

    cy = geo[1:2, :]
    R = _TP // _W_IMG
    pyh = (jax.lax.broadcasted_iota(jnp.int32, (R, 1), 0).astype(jnp.float32)
           + (t * R).astype(jnp.float32) + 0.5)
    dyh = pyh - cy
    k2 = row_ref[0:1, :] * dyh
    bh = (row_ref[1:2, :] * dyh) * dyh
    aw = aw_ref[...][None, :, :]
    u = aw + k2[:, None, :]
    power = bh[:, None, :] - u * u
    alpha = jnp.exp2(power).reshape(_TP, _N)
    acc = jax.lax.dot_general(
        geo[5:8, :], alpha, (((1,), (1,)), ((), ())),
        preferred_element_type=jnp.float32)
    o_ref[0] = jnp.clip(acc, 0.0, 1.0)


def kernel(inp, conv_w, conv_b, off_w1, off_b1, off_w2, off_b2,
           sc_w1, sc_b1, sc_w2, sc_b2, rot_w1, rot_b1, rot_w2, rot_b2,
           col_w1, col_b1, col_w2, col_b2, target_h, target_w):
    f32 = jnp.float32
    B, Cin, h_in, w_in = inp.shape
    C = conv_w.shape[0]

    x = jnp.transpose(inp, (0, 3, 2, 1))
    xp = jnp.pad(x, ((0, 0), (1, 1), (1, 1), (0, 0)))
    patches = jnp.concatenate(
        [xp[:, i:i + w_in, j:j + h_in, :] for i in range(3) for j in range(3)],
        axis=-1)
    K1 = 9 * Cin
    K1p = 32
    patches = patches.reshape(B, _N, K1)
    patches = jnp.pad(patches, ((0, 0), (0, 0), (0, K1p - K1)))
    wmat = jnp.transpose(conv_w, (3, 2, 1, 0)).reshape(K1, C)
    wmat = jnp.pad(wmat, ((0, K1p - K1), (0, 0)))

    w1o = off_w1.reshape(C, 9, _HID)
    w1s = sc_w1.reshape(C, 9, _HID)
    w1r = rot_w1.reshape(C, 9, _HID)
    w1c = col_w1.reshape(C, 9, _HID)
    b1 = jnp.concatenate([off_b1, sc_b1, rot_b1, col_b1]).reshape(1, 4 * _HID)
    b2 = jnp.concatenate([off_b2, sc_b2, rot_b2, col_b2]).reshape(8, 1)

    th_f = jnp.asarray(target_h, f32)
    tw_f = jnp.asarray(target_w, f32)
    factor = jnp.maximum(th_f / h_in, tw_f / w_in)
    off_factor = 2.0 * factor / jnp.maximum(th_f, tw_f)
    scal = jnp.stack([tw_f, th_f, 2.0 * factor, 3.0 * off_factor]).reshape(1, 4)

    r = 1.0 / _XG
    c1 = -1.0 + r + 2.0 * r * jnp.arange(_XG, dtype=f32)
    coord = jnp.stack(jnp.meshgrid(c1, c1, indexing='ij'), axis=-1)
    coordT = coord.reshape(_N, 2).T

    geo = pl.pallas_call(
        _encode_kernel,
        grid=(B,),
        in_specs=[
            pl.BlockSpec((1, _N, K1p), lambda b: (b, 0, 0)),
            pl.BlockSpec((K1p, C), lambda b: (0, 0)),
            pl.BlockSpec((1, C), lambda b: (0, 0)),
            pl.BlockSpec((C, 9, _HID), lambda b: (0, 0, 0)),
            pl.BlockSpec((C, 9, _HID), lambda b: (0, 0, 0)),
            pl.BlockSpec((C, 9, _HID), lambda b: (0, 0, 0)),
            pl.BlockSpec((C, 9, _HID), lambda b: (0, 0, 0)),
            pl.BlockSpec((1, 4 * _HID), lambda b: (0, 0)),
            pl.BlockSpec((_HID, 2), lambda b: (0, 0)),
            pl.BlockSpec((_HID, 2), lambda b: (0, 0)),
            pl.BlockSpec((_HID, 1), lambda b: (0, 0)),
            pl.BlockSpec((_HID, 3), lambda b: (0, 0)),
            pl.BlockSpec((8, 1), lambda b: (0, 0)),
            pl.BlockSpec((2, _N), lambda b: (0, 0)),
            pl.BlockSpec((1, 4), lambda b: (0, 0)),
        ],
        out_specs=pl.BlockSpec((1, 8, _N), lambda b: (b, 0, 0)),
        out_shape=jax.ShapeDtypeStruct((B, 8, _N), f32),
    )(patches, wmat, conv_b.reshape(1, C), w1o, w1s, w1r, w1c, b1,
      off_w2, sc_w2, rot_w2, col_w2, b2, coordT, scal)

    n_pix = _H_IMG * _W_IMG
    n_tiles = n_pix // _TP
    out = pl.pallas_call(
        _render_kernel,
        grid=(B, n_tiles),
        in_specs=[
            pl.BlockSpec((1, 8, _N), lambda b, t: (b, 0, 0)),
        ],
        out_specs=pl.BlockSpec((1, 3, _TP), lambda b, t: (b, 0, t)),
        out_shape=jax.ShapeDtypeStruct((B, 3, n_pix), f32),
        scratch_shapes=[
            pltpu.VMEM((_W_IMG, _N), f32),
            pltpu.VMEM((8, _N), f32),
        ],
    )(geo)

    return out.reshape(B, 3, _H_IMG, _W_IMG)
